# baseline (speedup 1.0000x reference)
import functools

import jax
import jax.numpy as jnp
from jax import lax
from jax.experimental import pallas as pl
from jax.experimental.pallas import tpu as pltpu
from jax.experimental.pallas import tpu_sc as plsc

B, N, D, K = 8, 2048, 128, 20
KPAD = 24
BR = 256


def _knn_body(xr_ref, xf_ref, idx_ref):
    b = pl.program_id(0)
    xr = xr_ref[0]
    xf = xf_ref[0]
    inner = lax.dot_general(xf, xr, (((1,), (1,)), ((), ())),
                            preferred_element_type=jnp.float32)
    sqr = jnp.sum(xr * xr, axis=1)
    sqf = jnp.sum(xf * xf, axis=1, keepdims=True)
    d = sqf - 2.0 * inner + sqr[None, :]
    iota_row = lax.broadcasted_iota(
        jnp.int32, (1, N), 1).astype(jnp.float32)
    lhs = jnp.concatenate(
        [iota_row, iota_row * iota_row, jnp.ones((1, N), jnp.float32)], axis=0)
    base = b * N
    idx0 = None
    m = jnp.full((1, BR), -jnp.inf, jnp.float32)
    use2 = jnp.zeros((1, BR), jnp.bool_)
    for k in range(K):
        cand = jnp.min(jnp.where(d > m, d, jnp.float32(jnp.inf)),
                       axis=0, keepdims=True)
        m = jnp.where(use2, m, cand)
        eq = d == m
        maskf = jnp.where(eq, 1.0, 0.0)
        mom = lax.dot_general(lhs, maskf, (((1,), (0,)), ((), ())),
                              precision=lax.Precision.HIGHEST,
                              preferred_element_type=jnp.float32)
        s1, s2, cnt = mom[0:1], mom[1:2], mom[2:3]
        delta = jnp.sqrt(jnp.maximum(cnt * s2 - s1 * s1, 0.0))
        i1 = jnp.clip((s1 - delta) / jnp.maximum(cnt, 1.0), 0.0, N - 1)
        idxf = jnp.clip(jnp.where(use2, s1 - i1, i1), 0.0, N - 1)
        idxk = idxf[0].astype(jnp.int32)
        idx_ref[0, k, :] = idxk + base
        if k == 0:
            idx0 = idxk + base
        use2 = jnp.logical_and(jnp.logical_not(use2), cnt >= 2.0)
    for k in range(K, KPAD):
        idx_ref[0, k, :] = idx0


def _knn_tc(x):
    return pl.pallas_call(
        _knn_body,
        grid=(B, N // BR),
        in_specs=[
            pl.BlockSpec((1, BR, D), lambda b, r: (b, r, 0)),
            pl.BlockSpec((1, N, D), lambda b, r: (b, 0, 0)),
        ],
        out_specs=pl.BlockSpec((1, KPAD, BR), lambda b, r: (b, 0, r)),
        out_shape=jax.ShapeDtypeStruct((B, KPAD, N), jnp.int32),
    )(x, x)


_P = 16
_NW = 32
_PPW = (B * N) // _NW
_NBLK = _PPW // _P


def _gather_max_sc(x_flat, idx_pm):
    mesh = plsc.VectorSubcoreMesh(core_axis_name="c", subcore_axis_name="s")

    @functools.partial(
        pl.kernel,
        out_type=jax.ShapeDtypeStruct((B * N, D), jnp.float32),
        mesh=mesh,
        scratch_types=[
            pltpu.VMEM((_P, KPAD), jnp.int32),
            pltpu.VMEM((_P, KPAD, D), jnp.float32),
            pltpu.VMEM((_P, D), jnp.float32),
            pltpu.SemaphoreType.DMA,
        ],
    )
    def k_fn(x_hbm, idx_hbm, out_hbm, idx_v, rows_v, out_v, sem):
        wid = lax.axis_index("s") * 2 + lax.axis_index("c")

        def block(j, _):
            pg = wid * _PPW + j * _P
            pltpu.sync_copy(idx_hbm.at[pl.ds(pg, _P)], idx_v)
            cps = []
            for p in range(_P):
                cp = pltpu.make_async_copy(
                    x_hbm.at[idx_v.at[p]], rows_v.at[p], sem)
                cp.start()
                cps.append(cp)
            for cp in cps:
                cp.wait()

            def row(p, _):
                for dc in range(D // 16):
                    sl = pl.ds(dc * 16, 16)
                    acc = rows_v[p, 0, sl]
                    for k in range(1, KPAD):
                        acc = jnp.maximum(acc, rows_v[p, k, sl])
                    out_v[p, sl] = acc
                return 0

            lax.fori_loop(0, _P, row, 0)
            pltpu.sync_copy(out_v, out_hbm.at[pl.ds(pg, _P)])
            return 0

        lax.fori_loop(0, _NBLK, block, 0)

    return k_fn(x_flat, idx_pm)


def _mlp_body(x_ref, mf_ref, w1a_ref, w1b_ref, b1_ref, g1_ref, be1_ref,
              w2_ref, b2_ref, g2_ref, be2_ref, wd_ref, bd_ref, out_ref):
    eps = 1e-3
    x = x_ref[...]
    mf = mf_ref[...]
    h = lax.dot_general(x, w1a_ref[...], (((1,), (0,)), ((), ())),
                        preferred_element_type=jnp.float32)
    h = h + lax.dot_general(mf - x, w1b_ref[...], (((1,), (0,)), ((), ())),
                            preferred_element_type=jnp.float32)
    h = jnp.maximum(h + b1_ref[...][None, :], 0.0)
    m1 = jnp.mean(h, axis=0, keepdims=True)
    v1 = jnp.mean(jnp.square(h - m1), axis=0, keepdims=True)
    h = g1_ref[...][None, :] * (h - m1) / jnp.sqrt(v1 + eps) + be1_ref[...][None, :]
    h = lax.dot_general(h, w2_ref[...], (((1,), (0,)), ((), ())),
                        preferred_element_type=jnp.float32)
    h = jnp.maximum(h + b2_ref[...][None, :], 0.0)
    m2 = jnp.mean(h, axis=0, keepdims=True)
    v2 = jnp.mean(jnp.square(h - m2), axis=0, keepdims=True)
    h = g2_ref[...][None, :] * (h - m2) / jnp.sqrt(v2 + eps) + be2_ref[...][None, :]
    pooled = jnp.stack(
        [jnp.max(h[bb * N:(bb + 1) * N], axis=0) for bb in range(B)])
    logits = lax.dot_general(pooled, wd_ref[...], (((1,), (0,)), ((), ())),
                             preferred_element_type=jnp.float32)
    logits = logits + bd_ref[...][None, :]
    mx = jnp.max(logits, axis=1, keepdims=True)
    e = jnp.exp(logits - mx)
    out_ref[...] = e / jnp.sum(e, axis=1, keepdims=True)


def _mlp_tc(x_flat, mf, W1a, W1b, b1, g1, be1, W2, b2, g2, be2, Wd, bd):
    return pl.pallas_call(
        _mlp_body,
        out_shape=jax.ShapeDtypeStruct((B, N), jnp.float32),
    )(x_flat, mf, W1a, W1b, b1, g1, be1, W2, b2, g2, be2, Wd, bd)


def kernel(inputs, W1, b1, g1, be1, W2, b2, g2, be2, Wd, bd):
    x = inputs
    idx = _knn_tc(x)
    x_flat = x.reshape(B * N, D)
    idx_pm = jnp.transpose(idx, (0, 2, 1)).reshape(B * N, KPAD)
    mf = _gather_max_sc(x_flat, idx_pm)
    W1a, W1b = W1[:D], W1[D:]
    return _mlp_tc(x_flat, mf, W1a, W1b, b1, g1, be1, W2, b2, g2, be2, Wd, bd)


# speedup vs baseline: 2.8832x; 2.8832x over previous
import functools

import jax
import jax.numpy as jnp
from jax import lax
from jax.experimental import pallas as pl
from jax.experimental.pallas import tpu as pltpu
from jax.experimental.pallas import tpu_sc as plsc

B, N, D, K = 8, 2048, 128, 20
KPAD = 24
BR = 256


def _knn_body(xr_ref, xf_ref, idx_ref):
    b = pl.program_id(0)
    xr = xr_ref[0]
    xf = xf_ref[0]
    inner = lax.dot_general(xf, xr, (((1,), (1,)), ((), ())),
                            preferred_element_type=jnp.float32)
    sqr = jnp.sum(xr * xr, axis=1)
    sqf = jnp.sum(xf * xf, axis=1, keepdims=True)
    d = sqf - 2.0 * inner + sqr[None, :]
    iota_i = lax.broadcasted_iota(jnp.int32, (1, N), 1)
    ih = (iota_i // 256).astype(jnp.float32)
    ia = ((iota_i // 16) % 16).astype(jnp.float32)
    ib = (iota_i % 16).astype(jnp.float32)
    lhs = jnp.concatenate(
        [ih, ia, ib, ih * ih, ia * ia, ib * ib, ih * ia, ih * ib, ia * ib,
         jnp.ones((1, N), jnp.float32)], axis=0)
    base = b * N
    idx0 = None
    m = jnp.full((1, BR), -jnp.inf, jnp.float32)
    use2 = jnp.zeros((1, BR), jnp.bool_)
    for k in range(K):
        cand = jnp.min(jnp.where(d > m, d, jnp.float32(jnp.inf)),
                       axis=0, keepdims=True)
        m = jnp.where(use2, m, cand)
        eq = d == m
        maskf = jnp.where(eq, 1.0, 0.0)
        mom = lax.dot_general(lhs, maskf, (((1,), (0,)), ((), ())),
                              preferred_element_type=jnp.float32)
        sh, sa, sb = mom[0:1], mom[1:2], mom[2:3]
        sh2, sa2, sb2 = mom[3:4], mom[4:5], mom[5:6]
        sha, shb, sab = mom[6:7], mom[7:8], mom[8:9]
        cnt = mom[9:10]
        s1 = 256.0 * sh + 16.0 * sa + sb
        s2 = (65536.0 * sh2 + 256.0 * sa2 + sb2
              + 8192.0 * sha + 512.0 * shb + 32.0 * sab)
        delta = jnp.sqrt(jnp.maximum(cnt * s2 - s1 * s1, 0.0))
        i1 = jnp.clip((s1 - delta) / jnp.maximum(cnt, 1.0), 0.0, N - 1)
        idxf = jnp.clip(jnp.where(use2, s1 - i1, i1), 0.0, N - 1)
        idxk = idxf[0].astype(jnp.int32)
        idx_ref[0, k, :] = idxk + base
        if k == 0:
            idx0 = idxk + base
        use2 = jnp.logical_and(jnp.logical_not(use2), cnt >= 2.0)
    for k in range(K, KPAD):
        idx_ref[0, k, :] = idx0


def _knn_tc(x):
    return pl.pallas_call(
        _knn_body,
        grid=(B, N // BR),
        in_specs=[
            pl.BlockSpec((1, BR, D), lambda b, r: (b, r, 0)),
            pl.BlockSpec((1, N, D), lambda b, r: (b, 0, 0)),
        ],
        out_specs=pl.BlockSpec((1, KPAD, BR), lambda b, r: (b, 0, r)),
        out_shape=jax.ShapeDtypeStruct((B, KPAD, N), jnp.int32),
    )(x, x)


_P = 16
_NW = 32
_PPW = (B * N) // _NW
_NBLK = _PPW // _P


def _gather_max_sc(x_flat, idx_pm):
    mesh = plsc.VectorSubcoreMesh(core_axis_name="c", subcore_axis_name="s")

    @functools.partial(
        pl.kernel,
        out_type=jax.ShapeDtypeStruct((B * N, D), jnp.float32),
        mesh=mesh,
        scratch_types=[
            pltpu.VMEM((_P, KPAD), jnp.int32),
            pltpu.VMEM((_P, KPAD, D), jnp.float32),
            pltpu.VMEM((_P, D), jnp.float32),
            pltpu.SemaphoreType.DMA,
        ],
    )
    def k_fn(x_hbm, idx_hbm, out_hbm, idx_v, rows_v, out_v, sem):
        wid = lax.axis_index("s") * 2 + lax.axis_index("c")

        def block(j, _):
            pg = wid * _PPW + j * _P
            pltpu.sync_copy(idx_hbm.at[pl.ds(pg, _P)], idx_v)
            cps = []
            for p in range(_P):
                cp = pltpu.make_async_copy(
                    x_hbm.at[idx_v.at[p]], rows_v.at[p], sem)
                cp.start()
                cps.append(cp)
            for cp in cps:
                cp.wait()

            def row(p, _):
                for dc in range(D // 16):
                    sl = pl.ds(dc * 16, 16)
                    acc = rows_v[p, 0, sl]
                    for k in range(1, KPAD):
                        acc = jnp.maximum(acc, rows_v[p, k, sl])
                    out_v[p, sl] = acc
                return 0

            lax.fori_loop(0, _P, row, 0)
            pltpu.sync_copy(out_v, out_hbm.at[pl.ds(pg, _P)])
            return 0

        lax.fori_loop(0, _NBLK, block, 0)

    return k_fn(x_flat, idx_pm)


def _mlp_body(x_ref, mf_ref, w1a_ref, w1b_ref, b1_ref, g1_ref, be1_ref,
              w2_ref, b2_ref, g2_ref, be2_ref, wd_ref, bd_ref, out_ref):
    eps = 1e-3
    x = x_ref[...]
    mf = mf_ref[...]
    h = lax.dot_general(x, w1a_ref[...], (((1,), (0,)), ((), ())),
                        preferred_element_type=jnp.float32)
    h = h + lax.dot_general(mf - x, w1b_ref[...], (((1,), (0,)), ((), ())),
                            preferred_element_type=jnp.float32)
    h = jnp.maximum(h + b1_ref[...][None, :], 0.0)
    m1 = jnp.mean(h, axis=0, keepdims=True)
    v1 = jnp.mean(jnp.square(h - m1), axis=0, keepdims=True)
    h = g1_ref[...][None, :] * (h - m1) / jnp.sqrt(v1 + eps) + be1_ref[...][None, :]
    h = lax.dot_general(h, w2_ref[...], (((1,), (0,)), ((), ())),
                        preferred_element_type=jnp.float32)
    h = jnp.maximum(h + b2_ref[...][None, :], 0.0)
    m2 = jnp.mean(h, axis=0, keepdims=True)
    v2 = jnp.mean(jnp.square(h - m2), axis=0, keepdims=True)
    h = g2_ref[...][None, :] * (h - m2) / jnp.sqrt(v2 + eps) + be2_ref[...][None, :]
    pooled = jnp.stack(
        [jnp.max(h[bb * N:(bb + 1) * N], axis=0) for bb in range(B)])
    logits = lax.dot_general(pooled, wd_ref[...], (((1,), (0,)), ((), ())),
                             preferred_element_type=jnp.float32)
    logits = logits + bd_ref[...][None, :]
    mx = jnp.max(logits, axis=1, keepdims=True)
    e = jnp.exp(logits - mx)
    out_ref[...] = e / jnp.sum(e, axis=1, keepdims=True)


def _mlp_tc(x_flat, mf, W1a, W1b, b1, g1, be1, W2, b2, g2, be2, Wd, bd):
    return pl.pallas_call(
        _mlp_body,
        out_shape=jax.ShapeDtypeStruct((B, N), jnp.float32),
    )(x_flat, mf, W1a, W1b, b1, g1, be1, W2, b2, g2, be2, Wd, bd)


def kernel(inputs, W1, b1, g1, be1, W2, b2, g2, be2, Wd, bd):
    x = inputs
    idx = _knn_tc(x)
    x_flat = x.reshape(B * N, D)
    idx_pm = jnp.transpose(idx, (0, 2, 1)).reshape(B * N, KPAD)
    mf = _gather_max_sc(x_flat, idx_pm)
    W1a, W1b = W1[:D], W1[D:]
    return _mlp_tc(x_flat, mf, W1a, W1b, b1, g1, be1, W2, b2, g2, be2, Wd, bd)


# speedup vs baseline: 3.4877x; 1.2097x over previous
import functools

import jax
import jax.numpy as jnp
from jax import lax
from jax.experimental import pallas as pl
from jax.experimental.pallas import tpu as pltpu
from jax.experimental.pallas import tpu_sc as plsc

B, N, D, K = 8, 2048, 128, 20
KPAD = 24
BR = 512


def _knn_body(xr_ref, xf_ref, idx_ref):
    b = pl.program_id(0)
    xr = xr_ref[0]
    xf = xf_ref[0]
    inner = lax.dot_general(xf, xr, (((1,), (1,)), ((), ())),
                            preferred_element_type=jnp.float32)
    sqr = jnp.sum(xr * xr, axis=1)
    sqf = jnp.sum(xf * xf, axis=1, keepdims=True)
    d = sqf - 2.0 * inner + sqr[None, :]
    iota_i = lax.broadcasted_iota(jnp.int32, (1, N), 1)
    ih = (iota_i // 256).astype(jnp.float32)
    ia = ((iota_i // 16) % 16).astype(jnp.float32)
    ib = (iota_i % 16).astype(jnp.float32)
    lhs = jnp.concatenate(
        [ih, ia, ib, ih * ih, ia * ia, ib * ib, ih * ia, ih * ib, ia * ib,
         jnp.ones((1, N), jnp.float32)], axis=0)
    base = b * N
    idx0 = None
    m = jnp.full((1, BR), -jnp.inf, jnp.float32)
    use2 = jnp.zeros((1, BR), jnp.bool_)
    for k in range(K):
        cand = jnp.min(jnp.where(d > m, d, jnp.float32(jnp.inf)),
                       axis=0, keepdims=True)
        m = jnp.where(use2, m, cand)
        eq = d == m
        maskf = jnp.where(eq, 1.0, 0.0)
        mom = lax.dot_general(lhs, maskf, (((1,), (0,)), ((), ())),
                              preferred_element_type=jnp.float32)
        sh, sa, sb = mom[0:1], mom[1:2], mom[2:3]
        sh2, sa2, sb2 = mom[3:4], mom[4:5], mom[5:6]
        sha, shb, sab = mom[6:7], mom[7:8], mom[8:9]
        cnt = mom[9:10]
        s1 = 256.0 * sh + 16.0 * sa + sb
        s2 = (65536.0 * sh2 + 256.0 * sa2 + sb2
              + 8192.0 * sha + 512.0 * shb + 32.0 * sab)
        delta = jnp.sqrt(jnp.maximum(cnt * s2 - s1 * s1, 0.0))
        i1 = jnp.clip((s1 - delta) / jnp.maximum(cnt, 1.0), 0.0, N - 1)
        idxf = jnp.clip(jnp.where(use2, s1 - i1, i1), 0.0, N - 1)
        idxk = idxf[0].astype(jnp.int32)
        idx_ref[0, k, :] = idxk + base
        if k == 0:
            idx0 = idxk + base
        use2 = jnp.logical_and(jnp.logical_not(use2), cnt >= 2.0)
    for k in range(K, KPAD):
        idx_ref[0, k, :] = idx0


def _knn_tc(x):
    return pl.pallas_call(
        _knn_body,
        grid=(B, N // BR),
        in_specs=[
            pl.BlockSpec((1, BR, D), lambda b, r: (b, r, 0)),
            pl.BlockSpec((1, N, D), lambda b, r: (b, 0, 0)),
        ],
        out_specs=pl.BlockSpec((1, KPAD, BR), lambda b, r: (b, 0, r)),
        out_shape=jax.ShapeDtypeStruct((B, KPAD, N), jnp.int32),
    )(x, x)


_P = 16
_NW = 32
_PPW = (B * N) // _NW
_NBLK = _PPW // _P


def _gather_max_sc(x_flat, idx_pm):
    mesh = plsc.VectorSubcoreMesh(core_axis_name="c", subcore_axis_name="s")

    @functools.partial(
        pl.kernel,
        out_type=jax.ShapeDtypeStruct((B * N, D), jnp.float32),
        mesh=mesh,
        scratch_types=[
            pltpu.VMEM((_P, KPAD), jnp.int32),
            pltpu.VMEM((_P, KPAD, D), jnp.float32),
            pltpu.VMEM((_P, D), jnp.float32),
            pltpu.SemaphoreType.DMA,
        ],
    )
    def k_fn(x_hbm, idx_hbm, out_hbm, idx_v, rows_v, out_v, sem):
        wid = lax.axis_index("s") * 2 + lax.axis_index("c")

        def block(j, _):
            pg = wid * _PPW + j * _P
            pltpu.sync_copy(idx_hbm.at[pl.ds(pg, _P)], idx_v)
            cps = []
            for p in range(_P):
                cp = pltpu.make_async_copy(
                    x_hbm.at[idx_v.at[p]], rows_v.at[p], sem)
                cp.start()
                cps.append(cp)
            for cp in cps:
                cp.wait()

            def row(p, _):
                for dc in range(D // 16):
                    sl = pl.ds(dc * 16, 16)
                    acc = rows_v[p, 0, sl]
                    for k in range(1, KPAD):
                        acc = jnp.maximum(acc, rows_v[p, k, sl])
                    out_v[p, sl] = acc
                return 0

            lax.fori_loop(0, _P, row, 0)
            pltpu.sync_copy(out_v, out_hbm.at[pl.ds(pg, _P)])
            return 0

        lax.fori_loop(0, _NBLK, block, 0)

    return k_fn(x_flat, idx_pm)


def _mlp_body(x_ref, mf_ref, w1a_ref, w1b_ref, b1_ref, g1_ref, be1_ref,
              w2_ref, b2_ref, g2_ref, be2_ref, wd_ref, bd_ref, out_ref):
    eps = 1e-3
    x = x_ref[...]
    mf = mf_ref[...]
    h = lax.dot_general(x, w1a_ref[...], (((1,), (0,)), ((), ())),
                        preferred_element_type=jnp.float32)
    h = h + lax.dot_general(mf - x, w1b_ref[...], (((1,), (0,)), ((), ())),
                            preferred_element_type=jnp.float32)
    h = jnp.maximum(h + b1_ref[...][None, :], 0.0)
    m1 = jnp.mean(h, axis=0, keepdims=True)
    v1 = jnp.mean(jnp.square(h - m1), axis=0, keepdims=True)
    h = g1_ref[...][None, :] * (h - m1) / jnp.sqrt(v1 + eps) + be1_ref[...][None, :]
    h = lax.dot_general(h, w2_ref[...], (((1,), (0,)), ((), ())),
                        preferred_element_type=jnp.float32)
    h = jnp.maximum(h + b2_ref[...][None, :], 0.0)
    m2 = jnp.mean(h, axis=0, keepdims=True)
    v2 = jnp.mean(jnp.square(h - m2), axis=0, keepdims=True)
    h = g2_ref[...][None, :] * (h - m2) / jnp.sqrt(v2 + eps) + be2_ref[...][None, :]
    pooled = jnp.stack(
        [jnp.max(h[bb * N:(bb + 1) * N], axis=0) for bb in range(B)])
    logits = lax.dot_general(pooled, wd_ref[...], (((1,), (0,)), ((), ())),
                             preferred_element_type=jnp.float32)
    logits = logits + bd_ref[...][None, :]
    mx = jnp.max(logits, axis=1, keepdims=True)
    e = jnp.exp(logits - mx)
    out_ref[...] = e / jnp.sum(e, axis=1, keepdims=True)


def _mlp_tc(x_flat, mf, W1a, W1b, b1, g1, be1, W2, b2, g2, be2, Wd, bd):
    return pl.pallas_call(
        _mlp_body,
        out_shape=jax.ShapeDtypeStruct((B, N), jnp.float32),
    )(x_flat, mf, W1a, W1b, b1, g1, be1, W2, b2, g2, be2, Wd, bd)


def kernel(inputs, W1, b1, g1, be1, W2, b2, g2, be2, Wd, bd):
    x = inputs
    idx = _knn_tc(x)
    x_flat = x.reshape(B * N, D)
    idx_pm = jnp.transpose(idx, (0, 2, 1)).reshape(B * N, KPAD)
    mf = _gather_max_sc(x_flat, idx_pm)
    W1a, W1b = W1[:D], W1[D:]
    return _mlp_tc(x_flat, mf, W1a, W1b, b1, g1, be1, W2, b2, g2, be2, Wd, bd)


# speedup vs baseline: 4.0156x; 1.1514x over previous
import functools

import jax
import jax.numpy as jnp
from jax import lax
from jax.experimental import pallas as pl
from jax.experimental.pallas import tpu as pltpu
from jax.experimental.pallas import tpu_sc as plsc

B, N, D, K = 8, 2048, 128, 20
KPAD = 24
BR = 512


def _knn_body(xr_ref, xf_ref, idx_ref):
    b = pl.program_id(0)
    xr = xr_ref[0]
    xf = xf_ref[0]
    inner = lax.dot_general(xf, xr, (((1,), (1,)), ((), ())),
                            preferred_element_type=jnp.float32)
    sqr = jnp.sum(xr * xr, axis=1)
    sqf = jnp.sum(xf * xf, axis=1, keepdims=True)
    d = sqf - 2.0 * inner + sqr[None, :]
    iota_i = lax.broadcasted_iota(jnp.int32, (1, N), 1)
    ih = (iota_i // 256).astype(jnp.float32)
    ia = ((iota_i // 16) % 16).astype(jnp.float32)
    ib = (iota_i % 16).astype(jnp.float32)
    lhs = jnp.concatenate(
        [ih, ia, ib, ih * ih, ia * ia, ib * ib, ih * ia, ih * ib, ia * ib,
         jnp.ones((1, N), jnp.float32)], axis=0)
    base = b * N
    idx0 = None
    m = jnp.full((1, BR), -jnp.inf, jnp.float32)
    use2 = jnp.zeros((1, BR), jnp.bool_)
    for k in range(K):
        cand = jnp.min(jnp.where(d > m, d, jnp.float32(jnp.inf)),
                       axis=0, keepdims=True)
        m = jnp.where(use2, m, cand)
        eq = d == m
        maskf = jnp.where(eq, 1.0, 0.0)
        mom = lax.dot_general(lhs, maskf, (((1,), (0,)), ((), ())),
                              preferred_element_type=jnp.float32)
        sh, sa, sb = mom[0:1], mom[1:2], mom[2:3]
        sh2, sa2, sb2 = mom[3:4], mom[4:5], mom[5:6]
        sha, shb, sab = mom[6:7], mom[7:8], mom[8:9]
        cnt = mom[9:10]
        s1 = 256.0 * sh + 16.0 * sa + sb
        s2 = (65536.0 * sh2 + 256.0 * sa2 + sb2
              + 8192.0 * sha + 512.0 * shb + 32.0 * sab)
        delta = jnp.sqrt(jnp.maximum(cnt * s2 - s1 * s1, 0.0))
        i1 = jnp.clip((s1 - delta) / jnp.maximum(cnt, 1.0), 0.0, N - 1)
        idxf = jnp.clip(jnp.where(use2, s1 - i1, i1), 0.0, N - 1)
        idxk = idxf[0].astype(jnp.int32)
        idx_ref[0, k, :] = idxk + base
        if k == 0:
            idx0 = idxk + base
        use2 = jnp.logical_and(jnp.logical_not(use2), cnt >= 2.0)
    for k in range(K, KPAD):
        idx_ref[0, k, :] = idx0


def _knn_tc(x):
    bh = x.shape[0]
    return pl.pallas_call(
        _knn_body,
        grid=(bh, N // BR),
        in_specs=[
            pl.BlockSpec((1, BR, D), lambda b, r: (b, r, 0)),
            pl.BlockSpec((1, N, D), lambda b, r: (b, 0, 0)),
        ],
        out_specs=pl.BlockSpec((1, KPAD, BR), lambda b, r: (b, 0, r)),
        out_shape=jax.ShapeDtypeStruct((bh, KPAD, N), jnp.int32),
    )(x, x)


_P = 32
_NW = 32


def _gather_max_sc(x_flat, idx_flat, bh):
    ppw = (bh * N) // _NW
    wpb = _NW // bh
    nblk = ppw // _P
    mesh = plsc.VectorSubcoreMesh(core_axis_name="c", subcore_axis_name="s")

    @functools.partial(
        pl.kernel,
        out_type=jax.ShapeDtypeStruct((bh * N, D), jnp.float32),
        mesh=mesh,
        scratch_types=[
            pltpu.VMEM((K, ppw), jnp.int32),
            pltpu.VMEM((K, _P, D), jnp.float32),
            pltpu.VMEM((_P, D), jnp.float32),
            pltpu.SemaphoreType.DMA,
            pltpu.SemaphoreType.DMA,
        ],
    )
    def k_fn(x_hbm, idx_hbm, out_hbm, idx_v, rows_v, out_v, sem, sem_i):
        wid = lax.axis_index("s") * 2 + lax.axis_index("c")
        b = wid // wpb
        i_base = (wid % wpb) * ppw
        icps = []
        for k in range(K):
            cp = pltpu.make_async_copy(
                idx_hbm.at[pl.ds((b * KPAD + k) * N + i_base, ppw)],
                idx_v.at[k], sem_i)
            cp.start()
            icps.append(cp)
        for cp in icps:
            cp.wait()

        def block(j, _):
            cps = []
            for k in range(K):
                cp = pltpu.make_async_copy(
                    x_hbm.at[idx_v.at[k, pl.ds(j * _P, _P)]],
                    rows_v.at[k], sem)
                cp.start()
                cps.append(cp)
            for cp in cps:
                cp.wait()

            def row(p, _):
                for dc in range(D // 16):
                    sl = pl.ds(dc * 16, 16)
                    acc = rows_v[0, p, sl]
                    for k in range(1, K):
                        acc = jnp.maximum(acc, rows_v[k, p, sl])
                    out_v[p, sl] = acc
                return 0

            lax.fori_loop(0, _P, row, 0)
            pltpu.sync_copy(
                out_v, out_hbm.at[pl.ds(b * N + i_base + j * _P, _P)])
            return 0

        lax.fori_loop(0, nblk, block, 0)

    return k_fn(x_flat, idx_flat)


def _mlp_body(x_ref, mf_ref, w1a_ref, w1b_ref, b1_ref, g1_ref, be1_ref,
              w2_ref, b2_ref, g2_ref, be2_ref, wd_ref, bd_ref, out_ref):
    eps = 1e-3
    x = x_ref[...]
    mf = mf_ref[...]
    h = lax.dot_general(x, w1a_ref[...], (((1,), (0,)), ((), ())),
                        preferred_element_type=jnp.float32)
    h = h + lax.dot_general(mf - x, w1b_ref[...], (((1,), (0,)), ((), ())),
                            preferred_element_type=jnp.float32)
    h = jnp.maximum(h + b1_ref[...][None, :], 0.0)
    m1 = jnp.mean(h, axis=0, keepdims=True)
    v1 = jnp.mean(jnp.square(h - m1), axis=0, keepdims=True)
    h = g1_ref[...][None, :] * (h - m1) / jnp.sqrt(v1 + eps) + be1_ref[...][None, :]
    h = lax.dot_general(h, w2_ref[...], (((1,), (0,)), ((), ())),
                        preferred_element_type=jnp.float32)
    h = jnp.maximum(h + b2_ref[...][None, :], 0.0)
    m2 = jnp.mean(h, axis=0, keepdims=True)
    v2 = jnp.mean(jnp.square(h - m2), axis=0, keepdims=True)
    h = g2_ref[...][None, :] * (h - m2) / jnp.sqrt(v2 + eps) + be2_ref[...][None, :]
    pooled = jnp.stack(
        [jnp.max(h[bb * N:(bb + 1) * N], axis=0) for bb in range(B)])
    logits = lax.dot_general(pooled, wd_ref[...], (((1,), (0,)), ((), ())),
                             preferred_element_type=jnp.float32)
    logits = logits + bd_ref[...][None, :]
    mx = jnp.max(logits, axis=1, keepdims=True)
    e = jnp.exp(logits - mx)
    out_ref[...] = e / jnp.sum(e, axis=1, keepdims=True)


def _mlp_tc(x_flat, mf, W1a, W1b, b1, g1, be1, W2, b2, g2, be2, Wd, bd):
    return pl.pallas_call(
        _mlp_body,
        out_shape=jax.ShapeDtypeStruct((B, N), jnp.float32),
    )(x_flat, mf, W1a, W1b, b1, g1, be1, W2, b2, g2, be2, Wd, bd)


def kernel(inputs, W1, b1, g1, be1, W2, b2, g2, be2, Wd, bd):
    x = inputs
    bh = B // 2
    mfs = []
    for h in range(2):
        xh = x[h * bh:(h + 1) * bh]
        idx = _knn_tc(xh)
        mfs.append(_gather_max_sc(
            xh.reshape(bh * N, D), idx.reshape(bh * KPAD * N), bh))
    x_flat = x.reshape(B * N, D)
    mf = jnp.concatenate(mfs, axis=0)
    W1a, W1b = W1[:D], W1[D:]
    return _mlp_tc(x_flat, mf, W1a, W1b, b1, g1, be1, W2, b2, g2, be2, Wd, bd)


# speedup vs baseline: 4.0725x; 1.0142x over previous
import functools

import jax
import jax.numpy as jnp
from jax import lax
from jax.experimental import pallas as pl
from jax.experimental.pallas import tpu as pltpu
from jax.experimental.pallas import tpu_sc as plsc

B, N, D, K = 8, 2048, 128, 20
KPAD = 24
BR = 512


def _knn_body(xr_ref, xf_ref, idx_ref):
    b = pl.program_id(0)
    xr = xr_ref[0]
    xf = xf_ref[0]
    inner = lax.dot_general(xf, xr, (((1,), (1,)), ((), ())),
                            preferred_element_type=jnp.float32)
    sqr = jnp.sum(xr * xr, axis=1)
    sqf = jnp.sum(xf * xf, axis=1, keepdims=True)
    d = sqf - 2.0 * inner + sqr[None, :]
    iota_i = lax.broadcasted_iota(jnp.int32, (1, N), 1)
    ih = (iota_i // 256).astype(jnp.float32)
    ia = ((iota_i // 16) % 16).astype(jnp.float32)
    ib = (iota_i % 16).astype(jnp.float32)
    lhs = jnp.concatenate(
        [ih, ia, ib, ih * ih, ia * ia, ib * ib, ih * ia, ih * ib, ia * ib,
         jnp.ones((1, N), jnp.float32)], axis=0)
    base = b * N
    idx0 = None
    m = jnp.full((1, BR), -jnp.inf, jnp.float32)
    use2 = jnp.zeros((1, BR), jnp.bool_)
    for k in range(K):
        cand = jnp.min(jnp.where(d > m, d, jnp.float32(jnp.inf)),
                       axis=0, keepdims=True)
        m = jnp.where(use2, m, cand)
        eq = d == m
        maskf = jnp.where(eq, 1.0, 0.0)
        mom = lax.dot_general(lhs, maskf, (((1,), (0,)), ((), ())),
                              preferred_element_type=jnp.float32)
        sh, sa, sb = mom[0:1], mom[1:2], mom[2:3]
        sh2, sa2, sb2 = mom[3:4], mom[4:5], mom[5:6]
        sha, shb, sab = mom[6:7], mom[7:8], mom[8:9]
        cnt = mom[9:10]
        s1 = 256.0 * sh + 16.0 * sa + sb
        s2 = (65536.0 * sh2 + 256.0 * sa2 + sb2
              + 8192.0 * sha + 512.0 * shb + 32.0 * sab)
        delta = jnp.sqrt(jnp.maximum(cnt * s2 - s1 * s1, 0.0))
        i1 = jnp.clip((s1 - delta) / jnp.maximum(cnt, 1.0), 0.0, N - 1)
        idxf = jnp.clip(jnp.where(use2, s1 - i1, i1), 0.0, N - 1)
        idxk = idxf[0].astype(jnp.int32)
        idx_ref[0, k, :] = idxk + base
        if k == 0:
            idx0 = idxk + base
        use2 = jnp.logical_and(jnp.logical_not(use2), cnt >= 2.0)
    for k in range(K, KPAD):
        idx_ref[0, k, :] = idx0


def _knn_tc(x):
    bh = x.shape[0]
    return pl.pallas_call(
        _knn_body,
        grid=(bh, N // BR),
        in_specs=[
            pl.BlockSpec((1, BR, D), lambda b, r: (b, r, 0)),
            pl.BlockSpec((1, N, D), lambda b, r: (b, 0, 0)),
        ],
        out_specs=pl.BlockSpec((1, KPAD, BR), lambda b, r: (b, 0, r)),
        out_shape=jax.ShapeDtypeStruct((bh, KPAD, N), jnp.int32),
    )(x, x)


_P = 32
_NW = 32


def _gather_max_sc(x_flat, idx_flat, bh):
    ppw = (bh * N) // _NW
    wpb = _NW // bh
    nblk = ppw // _P
    mesh = plsc.VectorSubcoreMesh(core_axis_name="c", subcore_axis_name="s")

    @functools.partial(
        pl.kernel,
        out_type=jax.ShapeDtypeStruct((bh * N, D), jnp.float32),
        mesh=mesh,
        scratch_types=[
            pltpu.VMEM((K, ppw), jnp.int32),
            pltpu.VMEM((K, _P, D), jnp.float32),
            pltpu.VMEM((_P, D), jnp.float32),
            pltpu.SemaphoreType.DMA,
            pltpu.SemaphoreType.DMA,
        ],
    )
    def k_fn(x_hbm, idx_hbm, out_hbm, idx_v, rows_v, out_v, sem, sem_i):
        wid = lax.axis_index("s") * 2 + lax.axis_index("c")
        b = wid // wpb
        i_base = (wid % wpb) * ppw
        icps = []
        for k in range(K):
            cp = pltpu.make_async_copy(
                idx_hbm.at[pl.ds((b * KPAD + k) * N + i_base, ppw)],
                idx_v.at[k], sem_i)
            cp.start()
            icps.append(cp)
        for cp in icps:
            cp.wait()

        def block(j, _):
            cps = []
            for k in range(K):
                cp = pltpu.make_async_copy(
                    x_hbm.at[idx_v.at[k, pl.ds(j * _P, _P)]],
                    rows_v.at[k], sem)
                cp.start()
                cps.append(cp)
            for cp in cps:
                cp.wait()

            def row(p, _):
                for dc in range(D // 16):
                    sl = pl.ds(dc * 16, 16)
                    acc = rows_v[0, p, sl]
                    for k in range(1, K):
                        acc = jnp.maximum(acc, rows_v[k, p, sl])
                    out_v[p, sl] = acc
                return 0

            lax.fori_loop(0, _P, row, 0)
            pltpu.sync_copy(
                out_v, out_hbm.at[pl.ds(b * N + i_base + j * _P, _P)])
            return 0

        lax.fori_loop(0, nblk, block, 0)

    return k_fn(x_flat, idx_flat)


def _mlp_body(x_ref, mf_ref, w1a_ref, w1b_ref, b1_ref, g1_ref, be1_ref,
              w2_ref, b2_ref, g2_ref, be2_ref, wd_ref, bd_ref, out_ref):
    eps = 1e-3
    x = x_ref[...]
    mf = mf_ref[...]
    h = lax.dot_general(x, w1a_ref[...], (((1,), (0,)), ((), ())),
                        preferred_element_type=jnp.float32)
    h = h + lax.dot_general(mf - x, w1b_ref[...], (((1,), (0,)), ((), ())),
                            preferred_element_type=jnp.float32)
    h = jnp.maximum(h + b1_ref[...][None, :], 0.0)
    m1 = jnp.mean(h, axis=0, keepdims=True)
    v1 = jnp.mean(jnp.square(h - m1), axis=0, keepdims=True)
    h = g1_ref[...][None, :] * (h - m1) / jnp.sqrt(v1 + eps) + be1_ref[...][None, :]
    h = lax.dot_general(h, w2_ref[...], (((1,), (0,)), ((), ())),
                        preferred_element_type=jnp.float32)
    h = jnp.maximum(h + b2_ref[...][None, :], 0.0)
    m2 = jnp.mean(h, axis=0, keepdims=True)
    v2 = jnp.mean(jnp.square(h - m2), axis=0, keepdims=True)
    h = g2_ref[...][None, :] * (h - m2) / jnp.sqrt(v2 + eps) + be2_ref[...][None, :]
    pooled = jnp.stack(
        [jnp.max(h[bb * N:(bb + 1) * N], axis=0) for bb in range(B)])
    logits = lax.dot_general(pooled, wd_ref[...], (((1,), (0,)), ((), ())),
                             preferred_element_type=jnp.float32)
    logits = logits + bd_ref[...][None, :]
    mx = jnp.max(logits, axis=1, keepdims=True)
    e = jnp.exp(logits - mx)
    out_ref[...] = e / jnp.sum(e, axis=1, keepdims=True)


def _mlp_tc(x_flat, mf, W1a, W1b, b1, g1, be1, W2, b2, g2, be2, Wd, bd):
    return pl.pallas_call(
        _mlp_body,
        out_shape=jax.ShapeDtypeStruct((B, N), jnp.float32),
    )(x_flat, mf, W1a, W1b, b1, g1, be1, W2, b2, g2, be2, Wd, bd)


def kernel(inputs, W1, b1, g1, be1, W2, b2, g2, be2, Wd, bd):
    x = inputs
    nchunk = 4
    bh = B // nchunk
    mfs = []
    for h in range(nchunk):
        xh = x[h * bh:(h + 1) * bh]
        idx = _knn_tc(xh)
        mfs.append(_gather_max_sc(
            xh.reshape(bh * N, D), idx.reshape(bh * KPAD * N), bh))
    x_flat = x.reshape(B * N, D)
    mf = jnp.concatenate(mfs, axis=0)
    W1a, W1b = W1[:D], W1[D:]
    return _mlp_tc(x_flat, mf, W1a, W1b, b1, g1, be1, W2, b2, g2, be2, Wd, bd)


# speedup vs baseline: 4.4837x; 1.1010x over previous
import functools

import jax
import jax.numpy as jnp
from jax import lax
from jax.experimental import pallas as pl
from jax.experimental.pallas import tpu as pltpu
from jax.experimental.pallas import tpu_sc as plsc

B, N, D, K = 8, 2048, 128, 20
KPAD = 24
BR = 512


def _knn_body(xr_ref, xf_ref, idx_ref):
    b = pl.program_id(0)
    xr = xr_ref[0]
    xf = xf_ref[0]
    inner = lax.dot_general(xf, xr, (((1,), (1,)), ((), ())),
                            preferred_element_type=jnp.float32)
    sqr = jnp.sum(xr * xr, axis=1)
    sqf = jnp.sum(xf * xf, axis=1, keepdims=True)
    d = sqf - 2.0 * inner + sqr[None, :]
    iota_i = lax.broadcasted_iota(jnp.int32, (1, N), 1)
    ih = (iota_i // 256).astype(jnp.float32)
    ia = ((iota_i // 16) % 16).astype(jnp.float32)
    ib = (iota_i % 16).astype(jnp.float32)
    lhs = jnp.concatenate(
        [ih, ia, ib, ih * ih, ia * ia, ib * ib, ih * ia, ih * ib, ia * ib,
         jnp.ones((1, N), jnp.float32)], axis=0)
    base = b * N
    idx0 = None

    def smallest4(w):
        h = w.shape[0] // 2
        a, bb = w[:h], w[h:]
        s1, s2 = jnp.minimum(a, bb), jnp.maximum(a, bb)
        h //= 2
        a1, b1 = s1[:h], s1[h:]
        a2, b2 = s2[:h], s2[h:]
        m1, x1 = jnp.minimum(a1, b1), jnp.maximum(a1, b1)
        m2, x2 = jnp.minimum(a2, b2), jnp.maximum(a2, b2)
        t = (m1, jnp.minimum(x1, m2), jnp.maximum(x1, m2), x2)
        while h > 1:
            h //= 2
            a = [u[:h] for u in t]
            bb = [u[h:] for u in t]
            L = [jnp.minimum(a[i], bb[3 - i]) for i in range(4)]
            p1, p3 = jnp.minimum(L[0], L[2]), jnp.maximum(L[0], L[2])
            p2, p4 = jnp.minimum(L[1], L[3]), jnp.maximum(L[1], L[3])
            t = (jnp.minimum(p1, p2), jnp.maximum(p1, p2),
                 jnp.minimum(p3, p4), jnp.maximum(p3, p4))
        return t

    def class_moments(val):
        maskf = jnp.where(d == val, 1.0, 0.0)
        mom = lax.dot_general(lhs, maskf, (((1,), (0,)), ((), ())),
                              preferred_element_type=jnp.float32)
        s1 = 256.0 * mom[0:1] + 16.0 * mom[1:2] + mom[2:3]
        s2 = (65536.0 * mom[3:4] + 256.0 * mom[4:5] + mom[5:6]
              + 8192.0 * mom[6:7] + 512.0 * mom[7:8] + 32.0 * mom[8:9])
        cnt = mom[9:10]
        delta = jnp.sqrt(jnp.maximum(cnt * s2 - s1 * s1, 0.0))
        i1 = jnp.clip((s1 - delta) / jnp.maximum(cnt, 1.0), 0.0, N - 1)
        return i1, s1, cnt

    m = jnp.full((1, BR), -jnp.inf, jnp.float32)
    use2 = jnp.zeros((1, BR), jnp.bool_)
    pend = jnp.zeros((1, BR), jnp.float32)
    for p in range(K // 4):
        v = smallest4(jnp.where(d > m, d, jnp.float32(jnp.inf)))
        cons = [jnp.where(use2, m, v[0]),
                jnp.where(use2, v[0], v[1]),
                jnp.where(use2, v[1], v[2]),
                jnp.where(use2, v[2], v[3])]
        idxs = []
        s1_r = cnt_r = None
        for r in range(4):
            i1, s1_r, cnt_r = class_moments(cons[r])
            if r == 0:
                idx = jnp.where(use2, pend, i1)
            else:
                idx = jnp.where(cons[r] == cons[r - 1],
                                s1_r - idxs[r - 1], i1)
            idx = jnp.clip(idx, 0.0, N - 1)
            idxs.append(idx)
            idxk = idx[0].astype(jnp.int32)
            idx_ref[0, 4 * p + r, :] = idxk + base
            if p == 0 and r == 0:
                idx0 = idxk + base
        m = cons[3]
        ec = (1.0 + jnp.where(cons[2] == m, 1.0, 0.0)
              + jnp.where(cons[1] == m, 1.0, 0.0)
              + jnp.where(cons[0] == m, 1.0, 0.0))
        use2 = cnt_r - ec >= 1.0
        pend = s1_r - idxs[3]
    for k in range(K, KPAD):
        idx_ref[0, k, :] = idx0


def _knn_tc(x):
    bh = x.shape[0]
    return pl.pallas_call(
        _knn_body,
        grid=(bh, N // BR),
        in_specs=[
            pl.BlockSpec((1, BR, D), lambda b, r: (b, r, 0)),
            pl.BlockSpec((1, N, D), lambda b, r: (b, 0, 0)),
        ],
        out_specs=pl.BlockSpec((1, KPAD, BR), lambda b, r: (b, 0, r)),
        out_shape=jax.ShapeDtypeStruct((bh, KPAD, N), jnp.int32),
    )(x, x)


_P = 32
_NW = 32


def _gather_max_sc(x_flat, idx_flat, bh):
    ppw = (bh * N) // _NW
    wpb = _NW // bh
    nblk = ppw // _P
    mesh = plsc.VectorSubcoreMesh(core_axis_name="c", subcore_axis_name="s")

    @functools.partial(
        pl.kernel,
        out_type=jax.ShapeDtypeStruct((bh * N, D), jnp.float32),
        mesh=mesh,
        scratch_types=[
            pltpu.VMEM((K, ppw), jnp.int32),
            pltpu.VMEM((K, _P, D), jnp.float32),
            pltpu.VMEM((_P, D), jnp.float32),
            pltpu.SemaphoreType.DMA,
            pltpu.SemaphoreType.DMA,
        ],
    )
    def k_fn(x_hbm, idx_hbm, out_hbm, idx_v, rows_v, out_v, sem, sem_i):
        wid = lax.axis_index("s") * 2 + lax.axis_index("c")
        b = wid // wpb
        i_base = (wid % wpb) * ppw
        icps = []
        for k in range(K):
            cp = pltpu.make_async_copy(
                idx_hbm.at[pl.ds((b * KPAD + k) * N + i_base, ppw)],
                idx_v.at[k], sem_i)
            cp.start()
            icps.append(cp)
        for cp in icps:
            cp.wait()

        def block(j, _):
            cps = []
            for k in range(K):
                cp = pltpu.make_async_copy(
                    x_hbm.at[idx_v.at[k, pl.ds(j * _P, _P)]],
                    rows_v.at[k], sem)
                cp.start()
                cps.append(cp)
            for cp in cps:
                cp.wait()

            def row(p, _):
                for dc in range(D // 16):
                    sl = pl.ds(dc * 16, 16)
                    acc = rows_v[0, p, sl]
                    for k in range(1, K):
                        acc = jnp.maximum(acc, rows_v[k, p, sl])
                    out_v[p, sl] = acc
                return 0

            lax.fori_loop(0, _P, row, 0)
            pltpu.sync_copy(
                out_v, out_hbm.at[pl.ds(b * N + i_base + j * _P, _P)])
            return 0

        lax.fori_loop(0, nblk, block, 0)

    return k_fn(x_flat, idx_flat)


def _mlp_body(x_ref, mf_ref, w1a_ref, w1b_ref, b1_ref, g1_ref, be1_ref,
              w2_ref, b2_ref, g2_ref, be2_ref, wd_ref, bd_ref, out_ref):
    eps = 1e-3
    x = x_ref[...]
    mf = mf_ref[...]
    h = lax.dot_general(x, w1a_ref[...], (((1,), (0,)), ((), ())),
                        preferred_element_type=jnp.float32)
    h = h + lax.dot_general(mf - x, w1b_ref[...], (((1,), (0,)), ((), ())),
                            preferred_element_type=jnp.float32)
    h = jnp.maximum(h + b1_ref[...][None, :], 0.0)
    m1 = jnp.mean(h, axis=0, keepdims=True)
    v1 = jnp.mean(jnp.square(h - m1), axis=0, keepdims=True)
    h = g1_ref[...][None, :] * (h - m1) / jnp.sqrt(v1 + eps) + be1_ref[...][None, :]
    h = lax.dot_general(h, w2_ref[...], (((1,), (0,)), ((), ())),
                        preferred_element_type=jnp.float32)
    h = jnp.maximum(h + b2_ref[...][None, :], 0.0)
    m2 = jnp.mean(h, axis=0, keepdims=True)
    v2 = jnp.mean(jnp.square(h - m2), axis=0, keepdims=True)
    h = g2_ref[...][None, :] * (h - m2) / jnp.sqrt(v2 + eps) + be2_ref[...][None, :]
    pooled = jnp.stack(
        [jnp.max(h[bb * N:(bb + 1) * N], axis=0) for bb in range(B)])
    logits = lax.dot_general(pooled, wd_ref[...], (((1,), (0,)), ((), ())),
                             preferred_element_type=jnp.float32)
    logits = logits + bd_ref[...][None, :]
    mx = jnp.max(logits, axis=1, keepdims=True)
    e = jnp.exp(logits - mx)
    out_ref[...] = e / jnp.sum(e, axis=1, keepdims=True)


def _mlp_tc(x_flat, mf, W1a, W1b, b1, g1, be1, W2, b2, g2, be2, Wd, bd):
    return pl.pallas_call(
        _mlp_body,
        out_shape=jax.ShapeDtypeStruct((B, N), jnp.float32),
    )(x_flat, mf, W1a, W1b, b1, g1, be1, W2, b2, g2, be2, Wd, bd)


def kernel(inputs, W1, b1, g1, be1, W2, b2, g2, be2, Wd, bd):
    x = inputs
    nchunk = 4
    bh = B // nchunk
    mfs = []
    for h in range(nchunk):
        xh = x[h * bh:(h + 1) * bh]
        idx = _knn_tc(xh)
        mfs.append(_gather_max_sc(
            xh.reshape(bh * N, D), idx.reshape(bh * KPAD * N), bh))
    x_flat = x.reshape(B * N, D)
    mf = jnp.concatenate(mfs, axis=0)
    W1a, W1b = W1[:D], W1[D:]
    return _mlp_tc(x_flat, mf, W1a, W1b, b1, g1, be1, W2, b2, g2, be2, Wd, bd)
